# baseline (speedup 1.0000x reference)
import jax
import jax.numpy as jnp
from jax import lax
from jax.experimental import pallas as pl
from jax.experimental.pallas import tpu as pltpu
from jax.experimental.pallas import tpu_sc as plsc

_N = 524288
_P = 25
_K = 8
_RMIN = -5.0
_RMAX = 5.0
_MINW = 0.001
_MINS = 0.001
_MAXS = 10.0
_NW = 32
_NSLAB = 4
_SLAB = _N // _NSLAB
_C = _SLAB // _NW
_G = _C // 16

_LN2 = 0.6931471805599453
_SQRT2 = 1.4142135623730951


def _log_f32(v):
    bits = lax.bitcast_convert_type(v, jnp.int32)
    e = lax.shift_right_logical(bits, 23) - 127
    m = lax.bitcast_convert_type(
        (bits & 0x7FFFFF) | 0x3F800000, jnp.float32)
    big = m > _SQRT2
    m = jnp.where(big, m * 0.5, m)
    e = jnp.where(big, e + 1, e)
    t = (m - 1.0) / (m + 1.0)
    t2 = t * t
    poly = 2.0 + t2 * (2.0 / 3.0 + t2 * (2.0 / 5.0 + t2 * (2.0 / 7.0
                       + t2 * (2.0 / 9.0))))
    return e.astype(jnp.float32) * _LN2 + t * poly


def _rqs_group(xv, p):
    sig = [1.0 / (1.0 + jnp.exp(-t)) for t in p]
    wsum = sig[0]
    for j in range(1, _K):
        wsum = wsum + sig[j]
    hsum = sig[_K]
    for j in range(_K + 1, 2 * _K):
        hsum = hsum + sig[j]
    rem = (_RMAX - _RMIN) - _K * _MINW
    wsc = rem / wsum
    hsc = rem / hsum
    w = [_MINW + sig[j] * wsc for j in range(_K)]
    h = [_MINW + sig[_K + j] * hsc for j in range(_K)]
    d = [_MINS + sig[2 * _K + j] * (_MAXS - _MINS) for j in range(_K + 1)]

    xk = [jnp.full((16,), _RMIN, jnp.float32)]
    yk = [jnp.full((16,), _RMIN, jnp.float32)]
    for j in range(_K):
        xk.append(xk[j] + w[j])
        yk.append(yk[j] + h[j])

    x_k, x_k1 = xk[0], xk[1]
    y_k, y_k1 = yk[0], yk[1]
    d_k, d_k1 = d[0], d[1]
    for j in range(1, _K):
        c = xk[j] <= xv
        x_k = jnp.where(c, xk[j], x_k)
        x_k1 = jnp.where(c, xk[j + 1], x_k1)
        y_k = jnp.where(c, yk[j], y_k)
        y_k1 = jnp.where(c, yk[j + 1], y_k1)
        d_k = jnp.where(c, d[j], d_k)
        d_k1 = jnp.where(c, d[j + 1], d_k1)

    width = x_k1 - x_k
    height = y_k1 - y_k
    inv_w = 1.0 / width
    xi = (xv - x_k) * inv_w
    s = height * inv_w
    xi1 = 1.0 - xi
    xx = xi * xi
    xx1 = xi * xi1
    x11 = xi1 * xi1
    num = s * xx + d_k * xx1
    den = s + (d_k1 + d_k - 2.0 * s) * xx1
    inv_den = 1.0 / den
    y_spline = y_k + height * num * inv_den
    deriv = (s * s) * (d_k1 * xx + 2.0 * s * xx1 + d_k * x11) * (
        inv_den * inv_den)

    below = xv <= _RMIN
    above = xv >= _RMAX
    y_out = jnp.where(below, _RMIN + (xv - _RMIN) * d[0],
                      jnp.where(above, _RMAX + (xv - _RMAX) * d[_K],
                                y_spline))
    dval = jnp.where(below, d[0], jnp.where(above, d[_K], deriv))
    return y_out, _log_f32(dval)


def _make_main_body(slab):
    def _main_body(x_hbm, pt0_hbm, pt3_hbm, y_hbm, part_hbm,
                   xv_ref, pv0_ref, pv3_ref, yv_ref, sv_ref):
        cid = lax.axis_index("c")
        sid = lax.axis_index("s")
        wid = sid * 2 + cid
        base = wid * _C
        pltpu.sync_copy(x_hbm.at[pl.ds(slab * _SLAB + base, _C)], xv_ref)
        pltpu.sync_copy(pt0_hbm.at[:, pl.ds(base, _C)], pv0_ref)
        pltpu.sync_copy(pt3_hbm.at[:, pl.ds(base, _C)], pv3_ref)

        @plsc.parallel_loop(0, _G, 1, unroll=2,
                            carry=jnp.zeros((16,), jnp.float32))
        def gbody(g, acc):
            row0 = g * 16
            xv = xv_ref[pl.ds(row0, 16)]
            p = [pv0_ref[j, pl.ds(row0, 16)] for j in range(24)]
            p.append(pv3_ref[0, pl.ds(row0, 16)])
            y_out, ld = _rqs_group(xv, p)
            yv_ref[pl.ds(row0, 16)] = y_out
            return acc + ld

        pltpu.sync_copy(yv_ref, y_hbm.at[pl.ds(base, _C)])
        sv_ref[...] = gbody
        pltpu.sync_copy(sv_ref, part_hbm.at[pl.ds(wid * 16, 16)])
    return _main_body


def _sum_body(part_hbm, out_hbm, buf_ref, ov_ref):
    cid = lax.axis_index("c")
    sid = lax.axis_index("s")
    wid = sid * 2 + cid

    @pl.when(wid == 0)
    def _():
        pltpu.sync_copy(part_hbm, buf_ref)

        def body(j, acc):
            return acc + buf_ref[pl.ds(j * 16, 16)]

        acc = lax.fori_loop(0, _NW * _NSLAB, body,
                            jnp.zeros((16,), jnp.float32))
        total = jnp.sum(acc)
        ov_ref[...] = jnp.broadcast_to(total, (16,))
        pltpu.sync_copy(ov_ref, out_hbm)


@jax.jit
def _impl(x, params):
    params_t = params.T
    mesh = plsc.VectorSubcoreMesh(core_axis_name="c", subcore_axis_name="s")
    ys = []
    parts = []
    for s in range(_NSLAB):
        main = pl.kernel(
            _make_main_body(s),
            out_type=(jax.ShapeDtypeStruct((_SLAB,), jnp.float32),
                      jax.ShapeDtypeStruct((_NW * 16,), jnp.float32)),
            mesh=mesh,
            scratch_types=(pltpu.VMEM((_C,), jnp.float32),
                           pltpu.VMEM((24, _C), jnp.float32),
                           pltpu.VMEM((1, _C), jnp.float32),
                           pltpu.VMEM((_C,), jnp.float32),
                           pltpu.VMEM((16,), jnp.float32)),
            compiler_params=pltpu.CompilerParams(needs_layout_passes=False),
            name=f"rqs_slab{s}",
        )
        lo = s * _SLAB
        y_s, part_s = main(x, params_t[0:24, lo:lo + _SLAB],
                           params_t[24:25, lo:lo + _SLAB])
        ys.append(y_s)
        parts.append(part_s)
    y = jnp.concatenate(ys)
    part = jnp.concatenate(parts)
    summ = pl.kernel(
        _sum_body,
        out_type=jax.ShapeDtypeStruct((16,), jnp.float32),
        mesh=plsc.VectorSubcoreMesh(core_axis_name="c",
                                    subcore_axis_name="s"),
        scratch_types=(pltpu.VMEM((_NW * 16 * _NSLAB,), jnp.float32),
                       pltpu.VMEM((16,), jnp.float32)),
        compiler_params=pltpu.CompilerParams(needs_layout_passes=False),
    )
    tot = summ(part)
    return y, tot[0]


def kernel(x, params):
    return _impl(x, params)


# speedup vs baseline: 1.2345x; 1.2345x over previous
import jax
import jax.numpy as jnp
from jax import lax
from jax.experimental import pallas as pl
from jax.experimental.pallas import tpu as pltpu
from jax.experimental.pallas import tpu_sc as plsc

_N = 524288
_P = 25
_K = 8
_RMIN = -5.0
_RMAX = 5.0
_MINW = 0.001
_MINS = 0.001
_MAXS = 10.0
_NW = 32
_EPW = _N // _NW
_C = 2048
_NCHUNK = _EPW // _C
_G = _C // 16

_LN2 = 0.6931471805599453
_SQRT2 = 1.4142135623730951


def _log_f32(v):
    bits = lax.bitcast_convert_type(v, jnp.int32)
    e = lax.shift_right_logical(bits, 23) - 127
    m = lax.bitcast_convert_type(
        (bits & 0x7FFFFF) | 0x3F800000, jnp.float32)
    big = m > _SQRT2
    m = jnp.where(big, m * 0.5, m)
    e = jnp.where(big, e + 1, e)
    t = (m - 1.0) / (m + 1.0)
    t2 = t * t
    poly = 2.0 + t2 * (2.0 / 3.0 + t2 * (2.0 / 5.0 + t2 * (2.0 / 7.0
                       + t2 * (2.0 / 9.0))))
    return e.astype(jnp.float32) * _LN2 + t * poly


def _rqs_group(xv, p):
    sig = [1.0 / (1.0 + jnp.exp(-t)) for t in p]
    wsum = sig[0]
    for j in range(1, _K):
        wsum = wsum + sig[j]
    hsum = sig[_K]
    for j in range(_K + 1, 2 * _K):
        hsum = hsum + sig[j]
    rem = (_RMAX - _RMIN) - _K * _MINW
    wsc = rem / wsum
    hsc = rem / hsum
    w = [_MINW + sig[j] * wsc for j in range(_K)]
    h = [_MINW + sig[_K + j] * hsc for j in range(_K)]
    d = [_MINS + sig[2 * _K + j] * (_MAXS - _MINS) for j in range(_K + 1)]

    xk = [jnp.full((16,), _RMIN, jnp.float32)]
    yk = [jnp.full((16,), _RMIN, jnp.float32)]
    for j in range(_K):
        xk.append(xk[j] + w[j])
        yk.append(yk[j] + h[j])

    x_k, x_k1 = xk[0], xk[1]
    y_k, y_k1 = yk[0], yk[1]
    d_k, d_k1 = d[0], d[1]
    for j in range(1, _K):
        c = xk[j] <= xv
        x_k = jnp.where(c, xk[j], x_k)
        x_k1 = jnp.where(c, xk[j + 1], x_k1)
        y_k = jnp.where(c, yk[j], y_k)
        y_k1 = jnp.where(c, yk[j + 1], y_k1)
        d_k = jnp.where(c, d[j], d_k)
        d_k1 = jnp.where(c, d[j + 1], d_k1)

    width = x_k1 - x_k
    height = y_k1 - y_k
    inv_w = 1.0 / width
    xi = (xv - x_k) * inv_w
    s = height * inv_w
    xi1 = 1.0 - xi
    xx = xi * xi
    xx1 = xi * xi1
    x11 = xi1 * xi1
    num = s * xx + d_k * xx1
    den = s + (d_k1 + d_k - 2.0 * s) * xx1
    inv_den = 1.0 / den
    y_spline = y_k + height * num * inv_den
    deriv = (s * s) * (d_k1 * xx + 2.0 * s * xx1 + d_k * x11) * (
        inv_den * inv_den)

    below = xv <= _RMIN
    above = xv >= _RMAX
    y_out = jnp.where(below, _RMIN + (xv - _RMIN) * d[0],
                      jnp.where(above, _RMAX + (xv - _RMAX) * d[_K],
                                y_spline))
    dval = jnp.where(below, d[0], jnp.where(above, d[_K], deriv))
    return y_out, _log_f32(dval)


def _main_body(x_hbm, pt0_hbm, pt3_hbm, y_hbm, part_hbm,
               xv0, xv1, pv0a, pv0b, pv3a, pv3b, yv0, yv1, sv_ref,
               sem_in0, sem_in1, sem_out0, sem_out1):
    cid = lax.axis_index("c")
    sid = lax.axis_index("s")
    wid = sid * 2 + cid
    xvs = (xv0, xv1)
    pv0s = (pv0a, pv0b)
    pv3s = (pv3a, pv3b)
    yvs = (yv0, yv1)
    sems_in = (sem_in0, sem_in1)
    sems_out = (sem_out0, sem_out1)

    def start_in(ch, b):
        base = wid * _EPW + ch * _C
        return (
            pltpu.async_copy(x_hbm.at[pl.ds(base, _C)], xvs[b], sems_in[b]),
            pltpu.async_copy(pt0_hbm.at[:, pl.ds(base, _C)], pv0s[b],
                             sems_in[b]),
            pltpu.async_copy(pt3_hbm.at[:, pl.ds(base, _C)], pv3s[b],
                             sems_in[b]),
        )

    pend = start_in(0, 0)
    out_pend = [None, None]
    acc = jnp.zeros((16,), jnp.float32)
    for ch in range(_NCHUNK):
        b = ch % 2
        for cp in pend:
            cp.wait()
        if ch + 1 < _NCHUNK:
            pend = start_in(ch + 1, 1 - b)
        if out_pend[b] is not None:
            out_pend[b].wait()
        xv_ref = xvs[b]
        pv0_ref = pv0s[b]
        pv3_ref = pv3s[b]
        yv_ref = yvs[b]

        @plsc.parallel_loop(0, _G, 1, unroll=2, carry=acc)
        def gbody(g, acc):
            row0 = g * 16
            xv = xv_ref[pl.ds(row0, 16)]
            p = [pv0_ref[j, pl.ds(row0, 16)] for j in range(24)]
            p.append(pv3_ref[0, pl.ds(row0, 16)])
            y_out, ld = _rqs_group(xv, p)
            yv_ref[pl.ds(row0, 16)] = y_out
            return acc + ld

        acc = gbody
        base = wid * _EPW + ch * _C
        out_pend[b] = pltpu.async_copy(yvs[b], y_hbm.at[pl.ds(base, _C)],
                                       sems_out[b])
    for op in out_pend:
        if op is not None:
            op.wait()
    sv_ref[...] = acc
    pltpu.sync_copy(sv_ref, part_hbm.at[pl.ds(wid * 16, 16)])


def _sum_body(part_hbm, out_hbm, buf_ref, ov_ref):
    cid = lax.axis_index("c")
    sid = lax.axis_index("s")
    wid = sid * 2 + cid

    @pl.when(wid == 0)
    def _():
        pltpu.sync_copy(part_hbm, buf_ref)

        def body(j, acc):
            return acc + buf_ref[pl.ds(j * 16, 16)]

        acc = lax.fori_loop(0, _NW, body, jnp.zeros((16,), jnp.float32))
        total = jnp.sum(acc)
        ov_ref[...] = jnp.broadcast_to(total, (16,))
        pltpu.sync_copy(ov_ref, out_hbm)


@jax.jit
def _impl(x, params):
    params_t = params.T
    mesh = plsc.VectorSubcoreMesh(core_axis_name="c", subcore_axis_name="s")
    main = pl.kernel(
        _main_body,
        out_type=(jax.ShapeDtypeStruct((_N,), jnp.float32),
                  jax.ShapeDtypeStruct((_NW * 16,), jnp.float32)),
        mesh=mesh,
        scratch_types=(pltpu.VMEM((_C,), jnp.float32),
                       pltpu.VMEM((_C,), jnp.float32),
                       pltpu.VMEM((24, _C), jnp.float32),
                       pltpu.VMEM((24, _C), jnp.float32),
                       pltpu.VMEM((1, _C), jnp.float32),
                       pltpu.VMEM((1, _C), jnp.float32),
                       pltpu.VMEM((_C,), jnp.float32),
                       pltpu.VMEM((_C,), jnp.float32),
                       pltpu.VMEM((16,), jnp.float32),
                       pltpu.SemaphoreType.DMA,
                       pltpu.SemaphoreType.DMA,
                       pltpu.SemaphoreType.DMA,
                       pltpu.SemaphoreType.DMA),
        compiler_params=pltpu.CompilerParams(needs_layout_passes=False),
    )
    y, part = main(x, params_t[0:24], params_t[24:25])
    summ = pl.kernel(
        _sum_body,
        out_type=jax.ShapeDtypeStruct((16,), jnp.float32),
        mesh=plsc.VectorSubcoreMesh(core_axis_name="c",
                                    subcore_axis_name="s"),
        scratch_types=(pltpu.VMEM((_NW * 16,), jnp.float32),
                       pltpu.VMEM((16,), jnp.float32)),
        compiler_params=pltpu.CompilerParams(needs_layout_passes=False),
    )
    tot = summ(part)
    return y, tot[0]


def kernel(x, params):
    return _impl(x, params)
